# baseline (speedup 1.0000x reference)
import functools

import jax
import jax.numpy as jnp
from jax import lax
from jax.experimental import pallas as pl
from jax.experimental.pallas import tpu as pltpu
from jax.experimental.pallas import tpu_sc as plsc

_NC = 2
_NS = 16
_NW = _NC * _NS
_BANDS = 8


def _copy_block(x_ref, o_ref):
    o_ref[...] = x_ref[...]


def _pipelined_copy(x, block_cols):
    rows, cols = x.shape
    return pl.pallas_call(
        _copy_block,
        grid=(pl.cdiv(cols, block_cols),),
        in_specs=[pl.BlockSpec((rows, block_cols), lambda i: (0, i))],
        out_specs=pl.BlockSpec((rows, block_cols), lambda i: (0, i)),
        out_shape=jax.ShapeDtypeStruct(x.shape, x.dtype),
    )(x)


_CW = 1536


def _sc_user_body(cols, u_hbm, u_out, buf):
    w = lax.axis_index("s") * _NC + lax.axis_index("c")
    rows = u_hbm.shape[0]
    n_full = cols // _CW

    def _move(col, width, off=0):
        src = u_hbm.at[:, pl.ds(col, width)]
        dst = u_out.at[:, pl.ds(col, width)]
        stage = buf.at[:, pl.ds(off, width)]
        pltpu.sync_copy(src, stage)
        pltpu.sync_copy(stage, dst)

    for k in range((n_full + _NW - 1) // _NW):
        c = k * _NW + w

        @pl.when(c < n_full)
        def _():
            _move(c * _CW, _CW)

    rem_tiles = (cols - n_full * _CW) // 128

    @pl.when(w == _NW - 1)
    def _():
        if rem_tiles:
            _move(n_full * _CW, rem_tiles * 128)


def _sc_copy(u_t):
    mesh = plsc.VectorSubcoreMesh(core_axis_name="c", subcore_axis_name="s")
    return functools.partial(
        pl.kernel,
        out_type=jax.ShapeDtypeStruct(u_t.shape, u_t.dtype),
        mesh=mesh,
        scratch_types=[
            pltpu.VMEM((u_t.shape[0], _CW), jnp.float32),
        ],
        compiler_params=pltpu.CompilerParams(use_tc_tiling_on_sc=True),
    )(functools.partial(_sc_user_body, u_t.shape[1]))(u_t)


def _patch_block(u_prev_ref, src_ref, out_ref):
    out_ref[...] = src_ref[...]


def _patch_tail(u_sc, u_t, block_cols):
    rows, cols = u_t.shape
    last = cols // block_cols
    spec = pl.BlockSpec((rows, block_cols), lambda i: (0, last))
    return pl.pallas_call(
        _patch_block,
        grid=(1,),
        in_specs=[spec, spec],
        out_specs=spec,
        out_shape=jax.ShapeDtypeStruct(u_t.shape, u_t.dtype),
        input_output_aliases={0: 0},
    )(u_sc, u_t)


def kernel(embedding_user, embedding_item):
    u_t = embedding_user.T
    i_t = embedding_item.T
    u_sc = _sc_copy(u_t)
    i_out = _pipelined_copy(i_t, 32768)
    u_out = _patch_tail(u_sc, u_t, 256)
    return (u_out.T, i_out.T)


# speedup vs baseline: 1.0989x; 1.0989x over previous
import functools

import jax
import jax.numpy as jnp
from jax import lax
from jax.experimental import pallas as pl
from jax.experimental.pallas import tpu as pltpu
from jax.experimental.pallas import tpu_sc as plsc

_NC = 2
_NS = 16
_NW = _NC * _NS
_BANDS = 8


def _copy_block(x_ref, o_ref):
    o_ref[...] = x_ref[...]


def _pipelined_copy(x, block_cols):
    rows, cols = x.shape
    return pl.pallas_call(
        _copy_block,
        grid=(pl.cdiv(cols, block_cols),),
        in_specs=[pl.BlockSpec((rows, block_cols), lambda i: (0, i))],
        out_specs=pl.BlockSpec((rows, block_cols), lambda i: (0, i)),
        out_shape=jax.ShapeDtypeStruct(x.shape, x.dtype),
    )(x)


_CW = 1536


def _sc_user_body(cols, u_hbm, u_out, buf):
    w = lax.axis_index("s") * _NC + lax.axis_index("c")
    rows = u_hbm.shape[0]
    n_full = cols // _CW

    def _move(col, width, off=0):
        src = u_hbm.at[:, pl.ds(col, width)]
        dst = u_out.at[:, pl.ds(col, width)]
        stage = buf.at[:, pl.ds(off, width)]
        pltpu.sync_copy(src, stage)
        pltpu.sync_copy(stage, dst)

    for k in range((n_full + _NW - 1) // _NW):
        c = k * _NW + w

        @pl.when(c < n_full)
        def _():
            _move(c * _CW, _CW)

    rem_tiles = (cols - n_full * _CW) // 128

    @pl.when(w == _NW - 1)
    def _():
        if rem_tiles:
            _move(n_full * _CW, rem_tiles * 128)


def _sc_copy(u_t):
    mesh = plsc.VectorSubcoreMesh(core_axis_name="c", subcore_axis_name="s")
    return functools.partial(
        pl.kernel,
        out_type=jax.ShapeDtypeStruct(u_t.shape, u_t.dtype),
        mesh=mesh,
        scratch_types=[
            pltpu.VMEM((u_t.shape[0], _CW), jnp.float32),
        ],
        compiler_params=pltpu.CompilerParams(use_tc_tiling_on_sc=True),
    )(functools.partial(_sc_user_body, u_t.shape[1]))(u_t)


def _patch_block(u_prev_ref, src_ref, out_ref):
    out_ref[...] = src_ref[...]


def _patch_tail(u_sc, u_t, block_cols):
    rows, cols = u_t.shape
    last = cols // block_cols
    spec = pl.BlockSpec((rows, block_cols), lambda i: (0, last))
    return pl.pallas_call(
        _patch_block,
        grid=(1,),
        in_specs=[spec, spec],
        out_specs=spec,
        out_shape=jax.ShapeDtypeStruct(u_t.shape, u_t.dtype),
        input_output_aliases={0: 0},
    )(u_sc, u_t)


def kernel(embedding_user, embedding_item):
    u_t = embedding_user.T
    i_t = embedding_item.T
    u_out = _pipelined_copy(u_t, 16384)
    i_out = _pipelined_copy(i_t, 49152)
    return (u_out.T, i_out.T)


# speedup vs baseline: 1.1113x; 1.0113x over previous
import jax
import jax.numpy as jnp
from jax.experimental import pallas as pl
from jax.experimental.pallas import tpu as pltpu


def _copy_block(x_ref, o_ref):
    o_ref[...] = x_ref[...]


def _pipelined_copy(x, block_cols):
    rows, cols = x.shape
    return pl.pallas_call(
        _copy_block,
        grid=(pl.cdiv(cols, block_cols),),
        in_specs=[pl.BlockSpec((rows, block_cols), lambda i: (0, i))],
        out_specs=pl.BlockSpec((rows, block_cols), lambda i: (0, i)),
        out_shape=jax.ShapeDtypeStruct(x.shape, x.dtype),
        compiler_params=pltpu.CompilerParams(vmem_limit_bytes=67108864),
    )(x)


def kernel(embedding_user, embedding_item):
    u_t = embedding_user.T
    i_t = embedding_item.T
    u_out = _pipelined_copy(u_t, 34816)
    i_out = _pipelined_copy(i_t, 62592)
    return (u_out.T, i_out.T)
